# Initial kernel scaffold; baseline (speedup 1.0000x reference)
import jax
import jax.numpy as jnp
from jax.experimental import pallas as pl


def kernel(x, edge_index, W1, b1, W2, b2):
    raise NotImplementedError("write your pallas kernel here")


# speedup vs baseline: 26.6887x; 26.6887x over previous
import functools

import jax
import jax.numpy as jnp
from jax import lax
from jax.experimental import pallas as pl
from jax.experimental.pallas import tpu as pltpu
from jax.experimental.pallas import tpu_sc as plsc

N = 10000
E = 320000
D_FEAT = 128
H1 = 32
H2 = 2

NC = 2
NS = 16
NW = NC * NS
CHUNK = 128
RPW = 80
EPAD = NW * RPW * CHUNK
NPAD = 10240
NPT = NPAD // NS

_mesh = plsc.VectorSubcoreMesh(
    core_axis_name="c", subcore_axis_name="s", num_cores=NC, num_subcores=NS)


def _worker(c, s):
  return s * NC + c


@functools.partial(
    pl.kernel,
    out_type=jax.ShapeDtypeStruct((NC, NPAD), jnp.float32),
    mesh=_mesh,
    scratch_types=[
        pltpu.VMEM((RPW, CHUNK), jnp.int32),
        pltpu.VMEM((CHUNK,), jnp.float32),
        pltpu.VMEM((NPT,), jnp.float32),
        pltpu.VMEM_SHARED((NPAD,), jnp.float32),
    ],
    compiler_params=pltpu.CompilerParams(use_tc_tiling_on_sc=False),
)
def _deg_kernel(dst_hbm, out_hbm, idx_v, ones_v, zbuf, acc_sh):
  c = lax.axis_index("c")
  s = lax.axis_index("s")
  wid = _worker(c, s)

  def _zero(i, _):
    zbuf[pl.ds(i * 16, 16)] = jnp.zeros((16,), jnp.float32)
    return _

  lax.fori_loop(0, NPT // 16, _zero, None)

  def _one(i, _):
    ones_v[pl.ds(i * 16, 16)] = jnp.ones((16,), jnp.float32)
    return _

  lax.fori_loop(0, CHUNK // 16, _one, None)

  pltpu.sync_copy(zbuf, acc_sh.at[pl.ds(s * NPT, NPT)])
  pltpu.sync_copy(dst_hbm.at[pl.ds(wid * RPW, RPW)], idx_v)
  plsc.subcore_barrier()

  def _scatter(j, _):
    pltpu.sync_copy(ones_v, acc_sh.at[idx_v.at[j]], add=True)
    return _

  lax.fori_loop(0, RPW, _scatter, None)
  plsc.subcore_barrier()
  pltpu.sync_copy(acc_sh.at[pl.ds(s * NPT, NPT)],
                  out_hbm.at[c, pl.ds(s * NPT, NPT)])


def _make_agg(W):
  @functools.partial(
      pl.kernel,
      out_type=jax.ShapeDtypeStruct((NC, NPAD, W), jnp.float32),
      mesh=_mesh,
      scratch_types=[
          pltpu.VMEM((RPW, CHUNK), jnp.int32),
          pltpu.VMEM((RPW, CHUNK), jnp.int32),
          pltpu.VMEM((CHUNK, W), jnp.float32),
          pltpu.VMEM_SHARED((NPAD, W), jnp.float32),
          pltpu.SemaphoreType.DMA,
      ],
      compiler_params=pltpu.CompilerParams(use_tc_tiling_on_sc=False),
  )
  def _agg(g_hbm, src_hbm, dst_hbm, zeros_hbm, out_hbm, sidx, didx, rows,
           acc_sh, sem):
    c = lax.axis_index("c")
    s = lax.axis_index("s")
    wid = _worker(c, s)

    @pl.when(c == 0)
    def _():
      pltpu.sync_copy(g_hbm.at[pl.ds(s * NPT, NPT)],
                      acc_sh.at[pl.ds(s * NPT, NPT)])

    @pl.when(c != 0)
    def _():
      pltpu.sync_copy(zeros_hbm, acc_sh.at[pl.ds(s * NPT, NPT)])

    pltpu.sync_copy(src_hbm.at[pl.ds(wid * RPW, RPW)], sidx)
    pltpu.sync_copy(dst_hbm.at[pl.ds(wid * RPW, RPW)], didx)
    plsc.subcore_barrier()

    def _edge_chunk(j, _):
      pltpu.async_copy(g_hbm.at[sidx.at[j]], rows, sem).wait()
      pltpu.sync_copy(rows, acc_sh.at[didx.at[j]], add=True)
      return _

    lax.fori_loop(0, RPW, _edge_chunk, None)
    plsc.subcore_barrier()
    pltpu.sync_copy(acc_sh.at[pl.ds(s * NPT, NPT)],
                    out_hbm.at[c, pl.ds(s * NPT, NPT)])

  return _agg


H2P = 8

_agg32 = _make_agg(H1)
_agg2 = _make_agg(H2P)


def _p1_body(deg_ref, x_ref, w1_ref, g1_ref, dinv_ref):
  deg = deg_ref[0, :N] + deg_ref[1, :N] + 1.0
  dinv = lax.rsqrt(deg)
  h0 = jnp.dot(x_ref[...], w1_ref[...], preferred_element_type=jnp.float32)
  g1_ref[:N, :] = h0 * dinv[:, None]
  g1_ref[N:, :] = jnp.zeros((NPAD - N, H1), jnp.float32)
  dinv_ref[...] = dinv[:, None]


def _p3_body(s1_ref, dinv_ref, b1_ref, w2p_ref, g2_ref):
  s1 = s1_ref[0, :N, :] + s1_ref[1, :N, :]
  dinv = dinv_ref[...]
  h1 = jnp.maximum(s1 * dinv + b1_ref[...], 0.0)
  g2_ref[:N, :] = jnp.dot(h1, w2p_ref[...],
                          preferred_element_type=jnp.float32) * dinv
  g2_ref[N:, :] = jnp.zeros((NPAD - N, H2P), jnp.float32)


def _p5_body(s2_ref, dinv_ref, b2_ref, out_ref):
  z = ((s2_ref[0, :N, :H2] + s2_ref[1, :N, :H2]) * dinv_ref[...]
       + b2_ref[...])
  m = jnp.max(z, axis=-1, keepdims=True)
  e = jnp.exp(z - m)
  out_ref[...] = e / jnp.sum(e, axis=-1, keepdims=True)


def kernel(x, edge_index, W1, b1, W2, b2):
  src = edge_index[0].astype(jnp.int32)
  dst = edge_index[1].astype(jnp.int32)
  npad_e = EPAD - E
  src2d = jnp.concatenate(
      [src, jnp.zeros((npad_e,), jnp.int32)]).reshape(EPAD // CHUNK, CHUNK)
  dst2d = jnp.concatenate(
      [dst, jnp.full((npad_e,), N, jnp.int32)]).reshape(EPAD // CHUNK, CHUNK)

  deg_parts = _deg_kernel(dst2d)

  g1, dinv = pl.pallas_call(
      _p1_body,
      out_shape=[
          jax.ShapeDtypeStruct((NPAD, H1), jnp.float32),
          jax.ShapeDtypeStruct((N, 1), jnp.float32),
      ],
  )(deg_parts, x, W1)

  z32 = jnp.zeros((NPT, H1), jnp.float32)
  z2 = jnp.zeros((NPT, H2P), jnp.float32)
  s1_parts = _agg32(g1, src2d, dst2d, z32)

  w2p = jnp.zeros((H1, H2P), jnp.float32).at[:, :H2].set(W2)
  g2 = pl.pallas_call(
      _p3_body,
      out_shape=jax.ShapeDtypeStruct((NPAD, H2P), jnp.float32),
  )(s1_parts, dinv, b1.reshape(1, H1), w2p)

  s2_parts = _agg2(g2, src2d, dst2d, z2)

  out = pl.pallas_call(
      _p5_body,
      out_shape=jax.ShapeDtypeStruct((N, H2), jnp.float32),
  )(s2_parts, dinv, b2.reshape(1, H2))
  return out


# speedup vs baseline: 33.6281x; 1.2600x over previous
import functools

import jax
import jax.numpy as jnp
from jax import lax
from jax.experimental import pallas as pl
from jax.experimental.pallas import tpu as pltpu
from jax.experimental.pallas import tpu_sc as plsc

N = 10000
E = 320000
D_FEAT = 128
H1 = 32
H2 = 2

NC = 2
NS = 16
NW = NC * NS
CHUNK = 128
RPW = 80
EPAD = NW * RPW * CHUNK
NPAD = 10240
NPT = NPAD // NS

_mesh = plsc.VectorSubcoreMesh(
    core_axis_name="c", subcore_axis_name="s", num_cores=NC, num_subcores=NS)


def _worker(c, s):
  return s * NC + c


@functools.partial(
    pl.kernel,
    out_type=jax.ShapeDtypeStruct((NC, NPAD), jnp.float32),
    mesh=_mesh,
    scratch_types=[
        pltpu.VMEM((RPW, CHUNK), jnp.int32),
        pltpu.VMEM((CHUNK,), jnp.float32),
        pltpu.VMEM((NPT,), jnp.float32),
        pltpu.VMEM_SHARED((NPAD,), jnp.float32),
    ],
    compiler_params=pltpu.CompilerParams(use_tc_tiling_on_sc=False),
)
def _deg_kernel(dst_hbm, out_hbm, idx_v, ones_v, zbuf, acc_sh):
  c = lax.axis_index("c")
  s = lax.axis_index("s")
  wid = _worker(c, s)

  def _zero(i, _):
    zbuf[pl.ds(i * 16, 16)] = jnp.zeros((16,), jnp.float32)
    return _

  lax.fori_loop(0, NPT // 16, _zero, None)

  def _one(i, _):
    ones_v[pl.ds(i * 16, 16)] = jnp.ones((16,), jnp.float32)
    return _

  lax.fori_loop(0, CHUNK // 16, _one, None)

  pltpu.sync_copy(zbuf, acc_sh.at[pl.ds(s * NPT, NPT)])
  pltpu.sync_copy(dst_hbm.at[pl.ds(wid * RPW, RPW)], idx_v)
  plsc.subcore_barrier()

  def _scatter(j, _):
    pltpu.sync_copy(ones_v, acc_sh.at[idx_v.at[j]], add=True)
    return _

  lax.fori_loop(0, RPW, _scatter, None)
  plsc.subcore_barrier()
  pltpu.sync_copy(acc_sh.at[pl.ds(s * NPT, NPT)],
                  out_hbm.at[c, pl.ds(s * NPT, NPT)])


def _make_agg(W):
  @functools.partial(
      pl.kernel,
      out_type=jax.ShapeDtypeStruct((NC, NPAD, W), jnp.float32),
      mesh=_mesh,
      scratch_types=[
          pltpu.VMEM((RPW, CHUNK), jnp.int32),
          pltpu.VMEM((RPW, CHUNK), jnp.int32),
          pltpu.VMEM((CHUNK, W), jnp.float32),
          pltpu.VMEM((CHUNK, W), jnp.float32),
          pltpu.VMEM_SHARED((NPAD, W), jnp.float32),
          pltpu.SemaphoreType.DMA,
          pltpu.SemaphoreType.DMA,
      ],
      compiler_params=pltpu.CompilerParams(use_tc_tiling_on_sc=False),
  )
  def _agg(g_hbm, src_hbm, dst_hbm, zeros_hbm, out_hbm, sidx, didx, rows0,
           rows1, acc_sh, sem0, sem1):
    c = lax.axis_index("c")
    s = lax.axis_index("s")
    wid = _worker(c, s)

    pltpu.sync_copy(zeros_hbm, acc_sh.at[pl.ds(s * NPT, NPT)])
    pltpu.sync_copy(src_hbm.at[pl.ds(wid * RPW, RPW)], sidx)
    pltpu.sync_copy(dst_hbm.at[pl.ds(wid * RPW, RPW)], didx)
    plsc.subcore_barrier()

    pltpu.async_copy(g_hbm.at[sidx.at[0]], rows0, sem0)
    pltpu.async_copy(g_hbm.at[sidx.at[1]], rows1, sem1)

    NIT = RPW // 2

    def _pair(j2, _):
      pltpu.make_async_copy(g_hbm.at[sidx.at[0]], rows0, sem0).wait()
      pltpu.sync_copy(rows0, acc_sh.at[didx.at[2 * j2]], add=True)

      @pl.when(j2 < NIT - 1)
      def _():
        pltpu.async_copy(g_hbm.at[sidx.at[2 * j2 + 2]], rows0, sem0)

      pltpu.make_async_copy(g_hbm.at[sidx.at[0]], rows1, sem1).wait()
      pltpu.sync_copy(rows1, acc_sh.at[didx.at[2 * j2 + 1]], add=True)

      @pl.when(j2 < NIT - 1)
      def _():
        pltpu.async_copy(g_hbm.at[sidx.at[2 * j2 + 3]], rows1, sem1)

      return _

    lax.fori_loop(0, NIT, _pair, None)
    plsc.subcore_barrier()
    pltpu.sync_copy(acc_sh.at[pl.ds(s * NPT, NPT)],
                    out_hbm.at[c, pl.ds(s * NPT, NPT)])

  return _agg


H2P = 8

_agg32 = _make_agg(H1)
_agg2 = _make_agg(H2P)


def _p1_body(deg_ref, x_ref, w1_ref, g1_ref, dinv_ref):
  deg = deg_ref[0, :N] + deg_ref[1, :N] + 1.0
  dinv = lax.rsqrt(deg)
  h0 = jnp.dot(x_ref[...], w1_ref[...], preferred_element_type=jnp.float32)
  g1_ref[:N, :] = h0 * dinv[:, None]
  g1_ref[N:, :] = jnp.zeros((NPAD - N, H1), jnp.float32)
  dinv_ref[...] = dinv[:, None]


def _p3_body(s1_ref, g1_ref, dinv_ref, b1_ref, w2p_ref, g2_ref):
  s1 = s1_ref[0, :N, :] + s1_ref[1, :N, :] + g1_ref[:N, :]
  dinv = dinv_ref[...]
  h1 = jnp.maximum(s1 * dinv + b1_ref[...], 0.0)
  g2_ref[:N, :] = jnp.dot(h1, w2p_ref[...],
                          preferred_element_type=jnp.float32) * dinv
  g2_ref[N:, :] = jnp.zeros((NPAD - N, H2P), jnp.float32)


def _p5_body(s2_ref, g2_ref, dinv_ref, b2_ref, out_ref):
  z = ((s2_ref[0, :N, :H2] + s2_ref[1, :N, :H2] + g2_ref[:N, :H2])
       * dinv_ref[...] + b2_ref[...])
  m = jnp.max(z, axis=-1, keepdims=True)
  e = jnp.exp(z - m)
  out_ref[...] = e / jnp.sum(e, axis=-1, keepdims=True)


def kernel(x, edge_index, W1, b1, W2, b2):
  src = edge_index[0].astype(jnp.int32)
  dst = edge_index[1].astype(jnp.int32)
  npad_e = EPAD - E
  src2d = jnp.concatenate(
      [src, jnp.zeros((npad_e,), jnp.int32)]).reshape(EPAD // CHUNK, CHUNK)
  dst2d = jnp.concatenate(
      [dst, jnp.full((npad_e,), N, jnp.int32)]).reshape(EPAD // CHUNK, CHUNK)

  deg_parts = _deg_kernel(dst2d)

  g1, dinv = pl.pallas_call(
      _p1_body,
      out_shape=[
          jax.ShapeDtypeStruct((NPAD, H1), jnp.float32),
          jax.ShapeDtypeStruct((N, 1), jnp.float32),
      ],
  )(deg_parts, x, W1)

  z32 = jnp.zeros((NPT, H1), jnp.float32)
  z2 = jnp.zeros((NPT, H2P), jnp.float32)
  s1_parts = _agg32(g1, src2d, dst2d, z32)

  w2p = jnp.zeros((H1, H2P), jnp.float32).at[:, :H2].set(W2)
  g2 = pl.pallas_call(
      _p3_body,
      out_shape=jax.ShapeDtypeStruct((NPAD, H2P), jnp.float32),
  )(s1_parts, g1, dinv, b1.reshape(1, H1), w2p)

  s2_parts = _agg2(g2, src2d, dst2d, z2)

  out = pl.pallas_call(
      _p5_body,
      out_shape=jax.ShapeDtypeStruct((N, H2), jnp.float32),
  )(s2_parts, g2, dinv, b2.reshape(1, H2))
  return out


# speedup vs baseline: 47.2860x; 1.4061x over previous
import functools

import jax
import jax.numpy as jnp
from jax import lax
from jax.experimental import pallas as pl
from jax.experimental.pallas import tpu as pltpu
from jax.experimental.pallas import tpu_sc as plsc

N = 10000
E = 320000
D_FEAT = 128
H1 = 32
H2 = 2

NC = 2
NS = 16
NW = NC * NS
CHUNK = 128
RPW = 80
EPAD = NW * RPW * CHUNK
NPAD = 10240
NPT = NPAD // NS

_mesh = plsc.VectorSubcoreMesh(
    core_axis_name="c", subcore_axis_name="s", num_cores=NC, num_subcores=NS)


def _worker(c, s):
  return s * NC + c


@functools.partial(
    pl.kernel,
    out_type=jax.ShapeDtypeStruct((NC, NPAD), jnp.float32),
    mesh=_mesh,
    scratch_types=[
        pltpu.VMEM((RPW, CHUNK), jnp.int32),
        pltpu.VMEM((CHUNK,), jnp.float32),
        pltpu.VMEM((NPT,), jnp.float32),
        pltpu.VMEM_SHARED((NPAD,), jnp.float32),
    ],
    compiler_params=pltpu.CompilerParams(use_tc_tiling_on_sc=False),
)
def _deg_kernel(dst_hbm, out_hbm, idx_v, ones_v, zbuf, acc_sh):
  c = lax.axis_index("c")
  s = lax.axis_index("s")
  wid = _worker(c, s)

  def _zero(i, _):
    zbuf[pl.ds(i * 16, 16)] = jnp.zeros((16,), jnp.float32)
    return _

  lax.fori_loop(0, NPT // 16, _zero, None)

  def _one(i, _):
    ones_v[pl.ds(i * 16, 16)] = jnp.ones((16,), jnp.float32)
    return _

  lax.fori_loop(0, CHUNK // 16, _one, None)

  pltpu.sync_copy(zbuf, acc_sh.at[pl.ds(s * NPT, NPT)])
  pltpu.sync_copy(dst_hbm.at[pl.ds(wid * RPW, RPW)], idx_v)
  plsc.subcore_barrier()

  def _scatter(j, _):
    pltpu.sync_copy(ones_v, acc_sh.at[idx_v.at[j]], add=True)
    return _

  lax.fori_loop(0, RPW, _scatter, None)
  plsc.subcore_barrier()
  pltpu.sync_copy(acc_sh.at[pl.ds(s * NPT, NPT)],
                  out_hbm.at[c, pl.ds(s * NPT, NPT)])


def _make_agg(W):
  @functools.partial(
      pl.kernel,
      out_type=jax.ShapeDtypeStruct((NC, NPAD, W), jnp.float32),
      mesh=_mesh,
      scratch_types=[
          pltpu.VMEM((RPW, CHUNK), jnp.int32),
          pltpu.VMEM((RPW, CHUNK), jnp.int32),
          pltpu.VMEM((CHUNK, W), jnp.float32),
          pltpu.VMEM((CHUNK, W), jnp.float32),
          pltpu.VMEM_SHARED((NPAD, W), jnp.float32),
          pltpu.SemaphoreType.DMA,
          pltpu.SemaphoreType.DMA,
      ],
      compiler_params=pltpu.CompilerParams(use_tc_tiling_on_sc=False),
  )
  def _agg(g_hbm, src_hbm, dst_hbm, zeros_hbm, out_hbm, sidx, didx, rows0,
           rows1, acc_sh, sem0, sem1):
    c = lax.axis_index("c")
    s = lax.axis_index("s")
    wid = _worker(c, s)

    pltpu.sync_copy(zeros_hbm, acc_sh.at[pl.ds(s * NPT, NPT)])
    pltpu.sync_copy(src_hbm.at[pl.ds(wid * RPW, RPW)], sidx)
    pltpu.sync_copy(dst_hbm.at[pl.ds(wid * RPW, RPW)], didx)
    plsc.subcore_barrier()

    pltpu.async_copy(g_hbm.at[sidx.at[0]], rows0, sem0)
    pltpu.async_copy(g_hbm.at[sidx.at[1]], rows1, sem1)

    NIT = RPW // 2

    def _pair(j2, _):
      pltpu.make_async_copy(g_hbm.at[sidx.at[0]], rows0, sem0).wait()
      pltpu.sync_copy(rows0, acc_sh.at[didx.at[2 * j2]], add=True)

      @pl.when(j2 < NIT - 1)
      def _():
        pltpu.async_copy(g_hbm.at[sidx.at[2 * j2 + 2]], rows0, sem0)

      pltpu.make_async_copy(g_hbm.at[sidx.at[0]], rows1, sem1).wait()
      pltpu.sync_copy(rows1, acc_sh.at[didx.at[2 * j2 + 1]], add=True)

      @pl.when(j2 < NIT - 1)
      def _():
        pltpu.async_copy(g_hbm.at[sidx.at[2 * j2 + 3]], rows1, sem1)

      return _

    lax.fori_loop(0, NIT, _pair, None)
    plsc.subcore_barrier()
    pltpu.sync_copy(acc_sh.at[pl.ds(s * NPT, NPT)],
                    out_hbm.at[c, pl.ds(s * NPT, NPT)])

  return _agg


H2P = 8

_agg32 = _make_agg(H1)
_agg2 = _make_agg(H2P)


def _p1_body(deg_ref, x_ref, w1_ref, g1_ref, dinv_ref):
  deg = deg_ref[0, :N] + deg_ref[1, :N] + 1.0
  dinv = lax.rsqrt(deg)
  h0 = jnp.dot(x_ref[...], w1_ref[...], preferred_element_type=jnp.float32)
  g1_ref[:N, :] = h0 * dinv[:, None]
  g1_ref[N:, :] = jnp.zeros((NPAD - N, H1), jnp.float32)
  dinv_ref[...] = dinv[:, None]


def _p3_body(s1_ref, g1_ref, dinv_ref, b1_ref, w2p_ref, g2_ref):
  s1 = s1_ref[0, :N, :] + s1_ref[1, :N, :] + g1_ref[:N, :]
  dinv = dinv_ref[...]
  h1 = jnp.maximum(s1 * dinv + b1_ref[...], 0.0)
  g2_ref[:N, :] = jnp.dot(h1, w2p_ref[...],
                          preferred_element_type=jnp.float32) * dinv
  g2_ref[N:, :] = jnp.zeros((NPAD - N, H2P), jnp.float32)


def _p5_body(s2_ref, g2_ref, dinv_ref, b2_ref, out_ref):
  z = ((s2_ref[0, :N, :H2] + s2_ref[1, :N, :H2] + g2_ref[:N, :H2])
       * dinv_ref[...] + b2_ref[...])
  m = jnp.max(z, axis=-1, keepdims=True)
  e = jnp.exp(z - m)
  out_ref[...] = e / jnp.sum(e, axis=-1, keepdims=True)


def kernel(x, edge_index, W1, b1, W2, b2):
  src = edge_index[0].astype(jnp.int32)
  dst = edge_index[1].astype(jnp.int32)
  npad_e = EPAD - E
  spread = jnp.arange(npad_e, dtype=jnp.int32) % (NPAD - N)
  src2d = jnp.concatenate([src, spread]).reshape(EPAD // CHUNK, CHUNK)
  dst2d = jnp.concatenate([dst, N + spread]).reshape(EPAD // CHUNK, CHUNK)

  deg_parts = _deg_kernel(dst2d)

  g1, dinv = pl.pallas_call(
      _p1_body,
      out_shape=[
          jax.ShapeDtypeStruct((NPAD, H1), jnp.float32),
          jax.ShapeDtypeStruct((N, 1), jnp.float32),
      ],
  )(deg_parts, x, W1)

  z32 = jnp.zeros((NPT, H1), jnp.float32)
  z2 = jnp.zeros((NPT, H2P), jnp.float32)
  s1_parts = _agg32(g1, src2d, dst2d, z32)

  w2p = jnp.zeros((H1, H2P), jnp.float32).at[:, :H2].set(W2)
  g2 = pl.pallas_call(
      _p3_body,
      out_shape=jax.ShapeDtypeStruct((NPAD, H2P), jnp.float32),
  )(s1_parts, g1, dinv, b1.reshape(1, H1), w2p)

  s2_parts = _agg2(g2, src2d, dst2d, z2)

  out = pl.pallas_call(
      _p5_body,
      out_shape=jax.ShapeDtypeStruct((N, H2), jnp.float32),
  )(s2_parts, g2, dinv, b2.reshape(1, H2))
  return out


# speedup vs baseline: 55.6551x; 1.1770x over previous
import functools

import jax
import jax.numpy as jnp
from jax import lax
from jax.experimental import pallas as pl
from jax.experimental.pallas import tpu as pltpu
from jax.experimental.pallas import tpu_sc as plsc

N = 10000
E = 320000
D_FEAT = 128
H1 = 32
H2 = 2

NC = 2
NS = 16
NW = NC * NS
CHUNK = 128
RPW = 80
EPAD = NW * RPW * CHUNK
NPAD = 10240
NPT = NPAD // NS

_mesh = plsc.VectorSubcoreMesh(
    core_axis_name="c", subcore_axis_name="s", num_cores=NC, num_subcores=NS)


def _worker(c, s):
  return s * NC + c


@functools.partial(
    pl.kernel,
    out_type=jax.ShapeDtypeStruct((NC, NPAD), jnp.float32),
    mesh=_mesh,
    scratch_types=[
        pltpu.VMEM((RPW, CHUNK), jnp.int32),
        pltpu.VMEM((CHUNK,), jnp.float32),
        pltpu.VMEM((NPT,), jnp.float32),
        pltpu.VMEM_SHARED((NPAD,), jnp.float32),
    ],
    compiler_params=pltpu.CompilerParams(use_tc_tiling_on_sc=False),
)
def _deg_kernel(dst_hbm, out_hbm, idx_v, ones_v, zbuf, acc_sh):
  c = lax.axis_index("c")
  s = lax.axis_index("s")
  wid = _worker(c, s)

  def _zero(i, _):
    zbuf[pl.ds(i * 16, 16)] = jnp.zeros((16,), jnp.float32)
    return _

  lax.fori_loop(0, NPT // 16, _zero, None)

  def _one(i, _):
    ones_v[pl.ds(i * 16, 16)] = jnp.ones((16,), jnp.float32)
    return _

  lax.fori_loop(0, CHUNK // 16, _one, None)

  pltpu.sync_copy(zbuf, acc_sh.at[pl.ds(s * NPT, NPT)])
  pltpu.sync_copy(dst_hbm.at[pl.ds(wid * RPW, RPW)], idx_v)
  plsc.subcore_barrier()

  def _scatter(j, _):
    pltpu.sync_copy(ones_v, acc_sh.at[idx_v.at[j]], add=True)
    return _

  lax.fori_loop(0, RPW, _scatter, None)
  plsc.subcore_barrier()
  pltpu.sync_copy(acc_sh.at[pl.ds(s * NPT, NPT)],
                  out_hbm.at[c, pl.ds(s * NPT, NPT)])


def _make_agg(W):
  @functools.partial(
      pl.kernel,
      out_type=jax.ShapeDtypeStruct((NC, NPAD, W), jnp.float32),
      mesh=_mesh,
      scratch_types=[
          pltpu.VMEM((RPW, CHUNK), jnp.int32),
          pltpu.VMEM((RPW, CHUNK), jnp.int32),
          pltpu.VMEM((CHUNK, W), jnp.float32),
          pltpu.VMEM((CHUNK, W), jnp.float32),
          pltpu.VMEM_SHARED((NPAD, W), jnp.float32),
          pltpu.SemaphoreType.DMA,
          pltpu.SemaphoreType.DMA,
      ],
      compiler_params=pltpu.CompilerParams(use_tc_tiling_on_sc=False),
  )
  def _agg(g_hbm, src_hbm, dst_hbm, zeros_hbm, out_hbm, sidx, didx, rows0,
           rows1, acc_sh, sem0, sem1):
    c = lax.axis_index("c")
    s = lax.axis_index("s")
    wid = _worker(c, s)

    @pl.when(c == 0)
    def _():
      pltpu.sync_copy(g_hbm.at[pl.ds(s * NPT, NPT)],
                      acc_sh.at[pl.ds(s * NPT, NPT)])

    @pl.when(c != 0)
    def _():
      pltpu.sync_copy(zeros_hbm, acc_sh.at[pl.ds(s * NPT, NPT)])

    pltpu.sync_copy(src_hbm.at[pl.ds(wid * RPW, RPW)], sidx)
    pltpu.sync_copy(dst_hbm.at[pl.ds(wid * RPW, RPW)], didx)
    plsc.subcore_barrier()

    pltpu.async_copy(g_hbm.at[sidx.at[0]], rows0, sem0)
    pltpu.async_copy(g_hbm.at[sidx.at[1]], rows1, sem1)

    NIT = RPW // 2

    def _pair(j2, _):
      pltpu.make_async_copy(g_hbm.at[sidx.at[0]], rows0, sem0).wait()
      pltpu.sync_copy(rows0, acc_sh.at[didx.at[2 * j2]], add=True)

      @pl.when(j2 < NIT - 1)
      def _():
        pltpu.async_copy(g_hbm.at[sidx.at[2 * j2 + 2]], rows0, sem0)

      pltpu.make_async_copy(g_hbm.at[sidx.at[0]], rows1, sem1).wait()
      pltpu.sync_copy(rows1, acc_sh.at[didx.at[2 * j2 + 1]], add=True)

      @pl.when(j2 < NIT - 1)
      def _():
        pltpu.async_copy(g_hbm.at[sidx.at[2 * j2 + 3]], rows1, sem1)

      return _

    lax.fori_loop(0, NIT, _pair, None)
    plsc.subcore_barrier()
    pltpu.sync_copy(acc_sh.at[pl.ds(s * NPT, NPT)],
                    out_hbm.at[c, pl.ds(s * NPT, NPT)])

  return _agg


H2P = 8

_agg32 = _make_agg(H1)
_agg2 = _make_agg(H2P)


R1P = NPAD * H1 // 128
R2P = NPAD * H2P // 128


def _p1_body(deg_ref, x_ref, w1_ref, g1_ref, dinv_ref):
  deg = deg_ref[0, :N] + deg_ref[1, :N] + 1.0
  dinv = lax.rsqrt(deg)
  h0 = jnp.dot(x_ref[...], w1_ref[...], preferred_element_type=jnp.float32)
  g1_ref[:N, :] = h0 * dinv[:, None]
  g1_ref[N:, :] = jnp.zeros((NPAD - N, H1), jnp.float32)
  dinv_ref[...] = dinv[:, None]


def _p3_body(s1_ref, dinvp_ref, b1p_ref, w2b_ref, dinv8_ref, g2_ref):
  z = (s1_ref[0] + s1_ref[1]) * dinvp_ref[...] + b1p_ref[...]
  h1 = jnp.maximum(z, 0.0)
  g2_ref[...] = jnp.dot(h1, w2b_ref[...],
                        preferred_element_type=jnp.float32) * dinv8_ref[...]


def _p5_body(s2_ref, dinv16_ref, b2p_ref, pswap_ref, out_ref):
  z = (s2_ref[0] + s2_ref[1]) * dinv16_ref[...] + b2p_ref[...]
  nb = jnp.dot(z, pswap_ref[...], preferred_element_type=jnp.float32,
               precision=lax.Precision.HIGHEST)
  m = jnp.maximum(z, nb)
  e = jnp.exp(z - m)
  esum = e + jnp.dot(e, pswap_ref[...], preferred_element_type=jnp.float32,
                     precision=lax.Precision.HIGHEST)
  out_ref[...] = e / esum


def kernel(x, edge_index, W1, b1, W2, b2):
  src = edge_index[0].astype(jnp.int32)
  dst = edge_index[1].astype(jnp.int32)
  npad_e = EPAD - E
  spread = jnp.arange(npad_e, dtype=jnp.int32) % (NPAD - N)
  src2d = jnp.concatenate([src, spread]).reshape(EPAD // CHUNK, CHUNK)
  dst2d = jnp.concatenate([dst, N + spread]).reshape(EPAD // CHUNK, CHUNK)

  deg_parts = _deg_kernel(dst2d)

  g1, dinv = pl.pallas_call(
      _p1_body,
      out_shape=[
          jax.ShapeDtypeStruct((NPAD, H1), jnp.float32),
          jax.ShapeDtypeStruct((N, 1), jnp.float32),
      ],
  )(deg_parts, x, W1)

  z32 = jnp.zeros((NPT, H1), jnp.float32)
  z2 = jnp.zeros((NPT, H2P), jnp.float32)
  s1_parts = _agg32(g1, src2d, dst2d, z32)

  dinv_pad = jnp.concatenate(
      [dinv, jnp.zeros((NPAD - N, 1), jnp.float32)])
  dinvp = jnp.broadcast_to(
      dinv_pad.reshape(R1P, 4, 1), (R1P, 4, H1)).reshape(R1P, 128)
  dinv8 = jnp.broadcast_to(
      dinv_pad.reshape(R1P, 4, 1), (R1P, 4, H2P)).reshape(R1P, 4 * H2P)
  dinv16 = jnp.broadcast_to(
      dinv_pad.reshape(R2P, 16, 1), (R2P, 16, H2P)).reshape(R2P, 128)
  b1p = jnp.tile(b1, 4).reshape(1, 128)
  w2p = jnp.zeros((H1, H2P), jnp.float32).at[:, :H2].set(W2)
  w2b = jnp.kron(jnp.eye(4, dtype=jnp.float32), w2p)
  b2p = jnp.tile(jnp.concatenate([b2, jnp.zeros((H2P - H2,), jnp.float32)]),
                 16).reshape(1, 128)
  lane0 = 8 * jnp.arange(16)
  pswap = (jnp.zeros((128, 128), jnp.float32)
           .at[lane0, lane0 + 1].set(1.0)
           .at[lane0 + 1, lane0].set(1.0))

  g2 = pl.pallas_call(
      _p3_body,
      out_shape=jax.ShapeDtypeStruct((R1P, 4 * H2P), jnp.float32),
  )(s1_parts.reshape(NC, R1P, 128), dinvp, b1p, w2b, dinv8)

  s2_parts = _agg2(g2.reshape(NPAD, H2P), src2d, dst2d, z2)

  outp = pl.pallas_call(
      _p5_body,
      out_shape=jax.ShapeDtypeStruct((R2P, 128), jnp.float32),
  )(s2_parts.reshape(NC, R2P, 128), dinv16, b2p, pswap)
  return outp.reshape(NPAD, H2P)[:N, :H2]
